# Initial kernel scaffold; baseline (speedup 1.0000x reference)
import jax
import jax.numpy as jnp
from jax import lax
from jax.experimental import pallas as pl
from jax.experimental.pallas import tpu as pltpu
from jax.experimental.pallas import tpu_sc as plsc

N_USERS = 25000
N_ITEMS = 25000
N = N_USERS + N_ITEMS
D = 64
DH = D // 2
N_LAYERS = 3
E = 800000
B = 4096

NC = 2
NS = 16
L = 16

CHUNK = 128
OUTER = 49
IB = 8
CT = OUTER * IB
CH = CT * NS
E_PAD = CH * CHUNK
ROWS_PER_TILE = N // NS
ZR = 125
BT = B // NS
UH = BT // CHUNK


def _body(emb_cat, srcp, dstp, wp, uix, iix,
          gamma_out, ping_a, ping_b,
          acc,
          src_blk, dst_blk, w_blk, rows, trows,
          u_idx, i_idx, u_acc, i_acc, zbuf, gamma_v):
    c = lax.axis_index("c")
    t = lax.axis_index("s")
    f32 = jnp.float32

    pltpu.sync_copy(uix.at[c, pl.ds(UH * t, UH)], u_idx)
    pltpu.sync_copy(iix.at[c, pl.ds(UH * t, UH)], i_idx)

    for h in range(UH):
        pltpu.sync_copy(emb_cat.at[u_idx.at[h]], u_acc.at[pl.ds(h * CHUNK, CHUNK)])
        pltpu.sync_copy(emb_cat.at[i_idx.at[h]], i_acc.at[pl.ds(h * CHUNK, CHUNK)])

    def _zb(r, _):
        z = jnp.zeros((L,), f32)
        zbuf[r, pl.ds(0, L)] = z
        zbuf[r, pl.ds(L, L)] = z
        return 0
    lax.fori_loop(0, ZR, _zb, 0)

    base_row = t * ROWS_PER_TILE
    cbase = t * CT

    def scale_rows(j):
        def _m(r, _):
            w = w_blk[j, r]
            wv = jnp.full((L,), w, dtype=f32)
            a = rows[r, pl.ds(0, L)]
            rows[r, pl.ds(0, L)] = a * wv
            b2 = rows[r, pl.ds(L, L)]
            rows[r, pl.ds(L, L)] = b2 * wv
            return 0
        lax.fori_loop(0, CHUNK, _m, 0)

    def acc_ui(dst_acc, idx):
        for h in range(UH):
            pltpu.sync_copy(acc.at[idx.at[h]], trows)

            def _a(b2, _):
                o = h * CHUNK + b2
                lo = dst_acc[o, pl.ds(0, L)] + trows[b2, pl.ds(0, L)]
                dst_acc[o, pl.ds(0, L)] = lo
                hi = dst_acc[o, pl.ds(L, L)] + trows[b2, pl.ds(L, L)]
                dst_acc[o, pl.ds(L, L)] = hi
                return 0
            lax.fori_loop(0, CHUNK, _a, 0)

    tables = [emb_cat, ping_a, ping_b]
    for k in range(N_LAYERS):
        table = tables[k]

        def _z(z, _):
            pltpu.sync_copy(zbuf, acc.at[pl.ds(base_row + z * ZR, ZR)])
            return 0
        lax.fori_loop(0, ROWS_PER_TILE // ZR, _z, 0)
        plsc.subcore_barrier()

        def _blk(ob, _):
            co = cbase + ob * IB
            pltpu.sync_copy(srcp.at[c, pl.ds(co, IB)], src_blk)
            pltpu.sync_copy(dstp.at[pl.ds(co, IB)], dst_blk)
            pltpu.sync_copy(wp.at[pl.ds(co, IB)], w_blk)
            for j in range(IB):
                pltpu.sync_copy(table.at[src_blk.at[j]], rows)
                scale_rows(j)
                pltpu.sync_copy(rows, acc.at[dst_blk.at[j]], add=True)
            return 0
        lax.fori_loop(0, OUTER, _blk, 0)
        plsc.subcore_barrier()

        acc_ui(u_acc, u_idx)
        acc_ui(i_acc, i_idx)
        if k < N_LAYERS - 1:
            out_tab = ping_a if k == 0 else ping_b
            pltpu.sync_copy(
                acc.at[pl.ds(base_row, ROWS_PER_TILE)],
                out_tab.at[pl.ds(c * N + base_row, ROWS_PER_TILE)])
        plsc.subcore_barrier()

    def _g(b2, _):
        s = (u_acc[b2, pl.ds(0, L)] * i_acc[b2, pl.ds(0, L)]
             + u_acc[b2, pl.ds(L, L)] * i_acc[b2, pl.ds(L, L)])
        gamma_v[b2] = jnp.sum(s) * f32(1.0 / 16.0)
        return 0
    lax.fori_loop(0, BT, _g, 0)
    pltpu.sync_copy(gamma_v, gamma_out.at[c, pl.ds(t * BT, BT)])


@jax.jit
def _sc_call(emb_cat, srcp, dstp, wp, uix, iix):
    mesh = plsc.VectorSubcoreMesh(core_axis_name="c", subcore_axis_name="s")
    f32 = jnp.float32
    call = pl.kernel(
        _body,
        out_type=[
            jax.ShapeDtypeStruct((NC, B), f32),
            jax.ShapeDtypeStruct((NC * N, DH), f32),
            jax.ShapeDtypeStruct((NC * N, DH), f32),
        ],
        mesh=mesh,
        scratch_types=[
            pltpu.VMEM_SHARED((N, DH), f32),
            pltpu.VMEM((IB, CHUNK), jnp.int32),
            pltpu.VMEM((IB, CHUNK), jnp.int32),
            pltpu.VMEM((IB, CHUNK), f32),
            pltpu.VMEM((CHUNK, DH), f32),
            pltpu.VMEM((CHUNK, DH), f32),
            pltpu.VMEM((UH, CHUNK), jnp.int32),
            pltpu.VMEM((UH, CHUNK), jnp.int32),
            pltpu.VMEM((BT, DH), f32),
            pltpu.VMEM((BT, DH), f32),
            pltpu.VMEM((ZR, DH), f32),
            pltpu.VMEM((BT,), f32),
        ],
    )
    return call(emb_cat, srcp, dstp, wp, uix, iix)


def kernel(user_emb, item_emb, edge_weight, users, items, edge_index):
    all_emb = jnp.concatenate([user_emb, item_emb], axis=0)
    emb_cat = jnp.concatenate([all_emb[:, :DH], all_emb[:, DH:]], axis=0)

    src = edge_index[0]
    dst = edge_index[1]
    pad = E_PAD - E
    srcp = jnp.pad(src, (0, pad))
    dstp = jnp.pad(dst, (0, pad)).reshape(CH, CHUNK)
    wp = jnp.pad(edge_weight, (0, pad)).reshape(CH, CHUNK)
    srcp = jnp.stack([srcp, srcp + N]).reshape(NC, CH, CHUNK)

    uix = jnp.stack([users, users + N]).reshape(NC, B // CHUNK, CHUNK)
    it = items + N_USERS
    iix = jnp.stack([it, it + N]).reshape(NC, B // CHUNK, CHUNK)

    gamma_parts, _, _ = _sc_call(emb_cat, srcp, dstp, wp, uix, iix)
    return gamma_parts[0] + gamma_parts[1]


# speedup vs baseline: 5.6274x; 5.6274x over previous
import jax
import jax.numpy as jnp
from jax import lax
from jax.experimental import pallas as pl
from jax.experimental.pallas import tpu as pltpu
from jax.experimental.pallas import tpu_sc as plsc

N_USERS = 25000
N_ITEMS = 25000
N = N_USERS + N_ITEMS
D = 64
DH = D // 2
N_LAYERS = 3
E = 800000
B = 4096

NC = 2
NS = 16
L = 16

CHUNK = 128
OUTER = 49
IB = 8
CT = OUTER * IB
CH = CT * NS
E_PAD = CH * CHUNK
NP = 50176
ROWS_PER_TILE = NP // NS
ZR = 112
BT = B // NS
UH = BT // CHUNK


def _body(emb_cat, srcp, dstp, wp, uix, iix,
          gamma_out, ping_a, ping_b,
          acc,
          src_blk, dst_blk, w_blk, rows,
          u_idx, i_idx, u_acc, i_acc, gamma_v):
    c = lax.axis_index("c")
    t = lax.axis_index("s")
    f32 = jnp.float32

    pltpu.sync_copy(uix.at[c, pl.ds(UH * t, UH)], u_idx)
    pltpu.sync_copy(iix.at[c, pl.ds(UH * t, UH)], i_idx)

    for h in range(UH):
        pltpu.sync_copy(emb_cat.at[u_idx.at[h]], u_acc.at[pl.ds(h * CHUNK, CHUNK)])
        pltpu.sync_copy(emb_cat.at[i_idx.at[h]], i_acc.at[pl.ds(h * CHUNK, CHUNK)])


    base_row = t * ROWS_PER_TILE
    cbase = t * CT

    def scale_rows(j):
        def _m(g, _):
            w16 = w_blk[j, pl.ds(g * L, L)]
            for lane in range(L):
                r = g * L + lane
                wv = jnp.full((L,), w16[lane], dtype=f32)
                rows[r, pl.ds(0, L)] = rows[r, pl.ds(0, L)] * wv
                rows[r, pl.ds(L, L)] = rows[r, pl.ds(L, L)] * wv
            return 0
        lax.fori_loop(0, CHUNK // L, _m, 0)

    def acc_ui(dst_acc, idx, table):
        for h in range(UH):
            pltpu.sync_copy(table.at[idx.at[h]], rows)

            def _a(b2, _):
                o = h * CHUNK + b2
                lo = dst_acc[o, pl.ds(0, L)] + rows[b2, pl.ds(0, L)]
                dst_acc[o, pl.ds(0, L)] = lo
                hi = dst_acc[o, pl.ds(L, L)] + rows[b2, pl.ds(L, L)]
                dst_acc[o, pl.ds(L, L)] = hi
                return 0
            lax.fori_loop(0, CHUNK, _a, 0)

    tables = [emb_cat, ping_a, ping_b]
    for k in range(N_LAYERS):
        table = tables[k]

        def _zb(r, _):
            z = jnp.zeros((L,), f32)
            rows[r, pl.ds(0, L)] = z
            rows[r, pl.ds(L, L)] = z
            return 0
        lax.fori_loop(0, ZR, _zb, 0)

        def _z(z, _):
            pltpu.sync_copy(rows.at[pl.ds(0, ZR)],
                            acc.at[pl.ds(base_row + z * ZR, ZR)])
            return 0
        lax.fori_loop(0, ROWS_PER_TILE // ZR, _z, 0)
        plsc.subcore_barrier()

        def _blk(ob, _):
            co = cbase + ob * IB
            pltpu.sync_copy(srcp.at[c, pl.ds(co, IB)], src_blk)
            pltpu.sync_copy(dstp.at[pl.ds(co, IB)], dst_blk)
            pltpu.sync_copy(wp.at[pl.ds(co, IB)], w_blk)
            for j in range(IB):
                pltpu.sync_copy(table.at[src_blk.at[j]], rows)
                scale_rows(j)
                pltpu.sync_copy(rows, acc.at[dst_blk.at[j]], add=True)
            return 0
        lax.fori_loop(0, OUTER, _blk, 0)
        plsc.subcore_barrier()

        out_tab = ping_a if k % 2 == 0 else ping_b
        pltpu.sync_copy(
            acc.at[pl.ds(base_row, ROWS_PER_TILE)],
            out_tab.at[pl.ds(c * NP + base_row, ROWS_PER_TILE)])
        plsc.subcore_barrier()

        acc_ui(u_acc, u_idx, out_tab)
        acc_ui(i_acc, i_idx, out_tab)

    lane_iota = lax.iota(jnp.int32, L)
    perms = [lane_iota ^ k for k in (8, 4, 2, 1)]

    def hsum(v):
        for p in perms:
            v = v + jnp.take(v, p)
        return v

    def _g(g, _):
        acc16 = jnp.zeros((L,), f32)
        for lane in range(L):
            b2 = g * L + lane
            s = (u_acc[b2, pl.ds(0, L)] * i_acc[b2, pl.ds(0, L)]
                 + u_acc[b2, pl.ds(L, L)] * i_acc[b2, pl.ds(L, L)])
            sv = hsum(s) * f32(1.0 / 16.0)
            acc16 = jnp.where(lane_iota == lane, sv, acc16)
        gamma_v[pl.ds(g * L, L)] = acc16
        return 0
    lax.fori_loop(0, BT // L, _g, 0)
    pltpu.sync_copy(gamma_v, gamma_out.at[c, pl.ds(t * BT, BT)])


@jax.jit
def _sc_call(emb_cat, srcp, dstp, wp, uix, iix):
    mesh = plsc.VectorSubcoreMesh(core_axis_name="c", subcore_axis_name="s")
    f32 = jnp.float32
    call = pl.kernel(
        _body,
        out_type=[
            jax.ShapeDtypeStruct((NC, B), f32),
            jax.ShapeDtypeStruct((NC * NP, DH), f32),
            jax.ShapeDtypeStruct((NC * NP, DH), f32),
        ],
        mesh=mesh,
        compiler_params=pltpu.CompilerParams(use_tc_tiling_on_sc=False),
        scratch_types=[
            pltpu.VMEM_SHARED((NP, DH), f32),
            pltpu.VMEM((IB, CHUNK), jnp.int32),
            pltpu.VMEM((IB, CHUNK), jnp.int32),
            pltpu.VMEM((IB, CHUNK), f32),
            pltpu.VMEM((CHUNK, DH), f32),
            pltpu.VMEM((UH, CHUNK), jnp.int32),
            pltpu.VMEM((UH, CHUNK), jnp.int32),
            pltpu.VMEM((BT, DH), f32),
            pltpu.VMEM((BT, DH), f32),
            pltpu.VMEM((BT,), f32),
        ],
    )
    return call(emb_cat, srcp, dstp, wp, uix, iix)


def kernel(user_emb, item_emb, edge_weight, users, items, edge_index):
    all_emb = jnp.concatenate([user_emb, item_emb], axis=0)
    zrows = jnp.zeros((NP - N, DH), jnp.float32)
    emb_cat = jnp.concatenate(
        [all_emb[:, :DH], zrows, all_emb[:, DH:], zrows], axis=0)

    src = edge_index[0]
    dst = edge_index[1]
    pad = E_PAD - E
    srcp = jnp.pad(src, (0, pad))
    dstp = jnp.pad(dst, (0, pad)).reshape(CH, CHUNK)
    wp = jnp.pad(edge_weight, (0, pad)).reshape(CH, CHUNK)
    srcp = jnp.stack([srcp, srcp + NP]).reshape(NC, CH, CHUNK)

    uix = jnp.stack([users, users + NP]).reshape(NC, B // CHUNK, CHUNK)
    it = items + N_USERS
    iix = jnp.stack([it, it + NP]).reshape(NC, B // CHUNK, CHUNK)

    gamma_parts, _, _ = _sc_call(emb_cat, srcp, dstp, wp, uix, iix)
    return gamma_parts[0] + gamma_parts[1]


# speedup vs baseline: 7.2034x; 1.2801x over previous
import jax
import jax.numpy as jnp
from jax import lax
from jax.experimental import pallas as pl
from jax.experimental.pallas import tpu as pltpu
from jax.experimental.pallas import tpu_sc as plsc

N_USERS = 25000
N_ITEMS = 25000
N = N_USERS + N_ITEMS
D = 64
DH = D // 2
N_LAYERS = 3
E = 800000
B = 4096

NC = 2
NS = 16
L = 16

CHUNK = 128
OUTER = 49
IB = 8
CT = OUTER * IB
CH = CT * NS
E_PAD = CH * CHUNK
NP = 50176
ROWS_PER_TILE = NP // NS
ZR = 112
BT = B // NS
UH = BT // CHUNK


def _body(emb_cat, srcp, dstp, wp, uix, iix,
          gamma_out, ping_a, ping_b,
          acc,
          src_blk, dst_blk, w_blk, rows, rows2,
          u_idx, i_idx, u_acc, i_acc, gamma_v, gsem):
    c = lax.axis_index("c")
    t = lax.axis_index("s")
    f32 = jnp.float32

    pltpu.sync_copy(uix.at[c, pl.ds(UH * t, UH)], u_idx)
    pltpu.sync_copy(iix.at[c, pl.ds(UH * t, UH)], i_idx)

    for h in range(UH):
        pltpu.sync_copy(emb_cat.at[u_idx.at[h]], u_acc.at[pl.ds(h * CHUNK, CHUNK)])
        pltpu.sync_copy(emb_cat.at[i_idx.at[h]], i_acc.at[pl.ds(h * CHUNK, CHUNK)])


    base_row = t * ROWS_PER_TILE
    cbase = t * CT

    def scale_rows(j, buf):
        def _m(g, _):
            w16 = w_blk[j, pl.ds(g * L, L)]
            for lane in range(L):
                r = g * L + lane
                wv = jnp.full((L,), w16[lane], dtype=f32)
                buf[r, pl.ds(0, L)] = buf[r, pl.ds(0, L)] * wv
                buf[r, pl.ds(L, L)] = buf[r, pl.ds(L, L)] * wv
            return 0
        lax.fori_loop(0, CHUNK // L, _m, 0)

    def acc_ui(dst_acc, idx, table):
        for h in range(UH):
            pltpu.sync_copy(table.at[idx.at[h]], rows)

            def _a(b2, _):
                o = h * CHUNK + b2
                lo = dst_acc[o, pl.ds(0, L)] + rows[b2, pl.ds(0, L)]
                dst_acc[o, pl.ds(0, L)] = lo
                hi = dst_acc[o, pl.ds(L, L)] + rows[b2, pl.ds(L, L)]
                dst_acc[o, pl.ds(L, L)] = hi
                return 0
            lax.fori_loop(0, CHUNK, _a, 0)

    tables = [emb_cat, ping_a, ping_b]
    for k in range(N_LAYERS):
        table = tables[k]

        def _zb(r, _):
            z = jnp.zeros((L,), f32)
            rows[r, pl.ds(0, L)] = z
            rows[r, pl.ds(L, L)] = z
            return 0
        lax.fori_loop(0, ZR, _zb, 0)

        def _z(z, _):
            pltpu.sync_copy(rows.at[pl.ds(0, ZR)],
                            acc.at[pl.ds(base_row + z * ZR, ZR)])
            return 0
        lax.fori_loop(0, ROWS_PER_TILE // ZR, _z, 0)
        plsc.subcore_barrier()

        def _blk(ob, _):
            co = cbase + ob * IB
            pltpu.sync_copy(srcp.at[c, pl.ds(co, IB)], src_blk)
            pltpu.sync_copy(dstp.at[pl.ds(co, IB)], dst_blk)
            pltpu.sync_copy(wp.at[pl.ds(co, IB)], w_blk)
            bufs = (rows, rows2)
            pend = pltpu.async_copy(table.at[src_blk.at[0]], bufs[0], gsem)
            for j in range(IB):
                buf = bufs[j % 2]
                pend.wait()
                if j + 1 < IB:
                    pend = pltpu.async_copy(
                        table.at[src_blk.at[j + 1]], bufs[(j + 1) % 2], gsem)
                scale_rows(j, buf)
                pltpu.sync_copy(buf, acc.at[dst_blk.at[j]], add=True)
            return 0
        lax.fori_loop(0, OUTER, _blk, 0)
        plsc.subcore_barrier()

        out_tab = ping_a if k % 2 == 0 else ping_b
        pltpu.sync_copy(
            acc.at[pl.ds(base_row, ROWS_PER_TILE)],
            out_tab.at[pl.ds(c * NP + base_row, ROWS_PER_TILE)])
        plsc.subcore_barrier()

        acc_ui(u_acc, u_idx, out_tab)
        acc_ui(i_acc, i_idx, out_tab)

    lane_iota = lax.iota(jnp.int32, L)
    perms = [lane_iota ^ k for k in (8, 4, 2, 1)]

    def hsum(v):
        for p in perms:
            v = v + jnp.take(v, p)
        return v

    def _g(g, _):
        acc16 = jnp.zeros((L,), f32)
        for lane in range(L):
            b2 = g * L + lane
            s = (u_acc[b2, pl.ds(0, L)] * i_acc[b2, pl.ds(0, L)]
                 + u_acc[b2, pl.ds(L, L)] * i_acc[b2, pl.ds(L, L)])
            sv = hsum(s) * f32(1.0 / 16.0)
            acc16 = jnp.where(lane_iota == lane, sv, acc16)
        gamma_v[pl.ds(g * L, L)] = acc16
        return 0
    lax.fori_loop(0, BT // L, _g, 0)
    pltpu.sync_copy(gamma_v, gamma_out.at[c, pl.ds(t * BT, BT)])


@jax.jit
def _sc_call(emb_cat, srcp, dstp, wp, uix, iix):
    mesh = plsc.VectorSubcoreMesh(core_axis_name="c", subcore_axis_name="s")
    f32 = jnp.float32
    call = pl.kernel(
        _body,
        out_type=[
            jax.ShapeDtypeStruct((NC, B), f32),
            jax.ShapeDtypeStruct((NC * NP, DH), f32),
            jax.ShapeDtypeStruct((NC * NP, DH), f32),
        ],
        mesh=mesh,
        compiler_params=pltpu.CompilerParams(use_tc_tiling_on_sc=False),
        scratch_types=[
            pltpu.VMEM_SHARED((NP, DH), f32),
            pltpu.VMEM((IB, CHUNK), jnp.int32),
            pltpu.VMEM((IB, CHUNK), jnp.int32),
            pltpu.VMEM((IB, CHUNK), f32),
            pltpu.VMEM((CHUNK, DH), f32),
            pltpu.VMEM((CHUNK, DH), f32),
            pltpu.VMEM((UH, CHUNK), jnp.int32),
            pltpu.VMEM((UH, CHUNK), jnp.int32),
            pltpu.VMEM((BT, DH), f32),
            pltpu.VMEM((BT, DH), f32),
            pltpu.VMEM((BT,), f32),
            pltpu.SemaphoreType.DMA,
        ],
    )
    return call(emb_cat, srcp, dstp, wp, uix, iix)


def kernel(user_emb, item_emb, edge_weight, users, items, edge_index):
    all_emb = jnp.concatenate([user_emb, item_emb], axis=0)
    zrows = jnp.zeros((NP - N, DH), jnp.float32)
    emb_cat = jnp.concatenate(
        [all_emb[:, :DH], zrows, all_emb[:, DH:], zrows], axis=0)

    src = edge_index[0]
    dst = edge_index[1]
    pad = E_PAD - E
    srcp = jnp.pad(src, (0, pad))
    dstp = jnp.pad(dst, (0, pad)).reshape(CH, CHUNK)
    wp = jnp.pad(edge_weight, (0, pad)).reshape(CH, CHUNK)
    srcp = jnp.stack([srcp, srcp + NP]).reshape(NC, CH, CHUNK)

    uix = jnp.stack([users, users + NP]).reshape(NC, B // CHUNK, CHUNK)
    it = items + N_USERS
    iix = jnp.stack([it, it + NP]).reshape(NC, B // CHUNK, CHUNK)

    gamma_parts, _, _ = _sc_call(emb_cat, srcp, dstp, wp, uix, iix)
    return gamma_parts[0] + gamma_parts[1]
